# Initial kernel scaffold; baseline (speedup 1.0000x reference)
import jax
import jax.numpy as jnp
from jax.experimental import pallas as pl


def kernel(x_user, x_item, ei_user_item, ei_item_user, Wl_ui, bl_ui, Wr_ui, br_ui, att_ui, bias_ui, Wl_iu, bl_iu, Wr_iu, br_iu, att_iu, bias_iu):
    raise NotImplementedError("write your pallas kernel here")


# speedup vs baseline: 10.2180x; 10.2180x over previous
import functools

import jax
import jax.numpy as jnp
from jax import lax
from jax.experimental import pallas as pl
from jax.experimental.pallas import tpu as pltpu
from jax.experimental.pallas import tpu_sc as plsc

N = 10000
D = 128
C = 128
E = 320000

NC = 2
NS = 16
L = 16

B = 80
NBLK = E // B
N_PAD = 10240
ROWS_PER_TEC = N_PAD // NS
RCHUNK = 16
DCHUNK = 128

_TC_BLOCK = 1000


def _tc_body(xu, xi, wlui, blui, wrui, brui, attui, wliu, bliu, wriu, briu,
             attiu, hs_ui, hd_ui, hs_iu, hd_iu, m_ui, m_iu, mx):
    step = pl.program_id(0)

    @pl.when(step == 0)
    def _():
        for i in range(4):
            mx[i] = 0.0

    a = jnp.dot(xu[...], wlui[...], preferred_element_type=jnp.float32) + blui[...]
    b = jnp.dot(xi[...], wrui[...], preferred_element_type=jnp.float32) + brui[...]
    c = jnp.dot(xi[...], wliu[...], preferred_element_type=jnp.float32) + bliu[...]
    d = jnp.dot(xu[...], wriu[...], preferred_element_type=jnp.float32) + briu[...]
    hs_ui[...] = a
    hd_ui[...] = b
    hs_iu[...] = c
    hd_iu[...] = d
    mx[0] = jnp.maximum(mx[0], jnp.max(jnp.sum(a * a, axis=1)))
    mx[1] = jnp.maximum(mx[1], jnp.max(jnp.sum(b * b, axis=1)))
    mx[2] = jnp.maximum(mx[2], jnp.max(jnp.sum(c * c, axis=1)))
    mx[3] = jnp.maximum(mx[3], jnp.max(jnp.sum(d * d, axis=1)))

    @pl.when(step == pl.num_programs(0) - 1)
    def _():
        nat_ui = jnp.sqrt(jnp.sum(attui[...] ** 2))
        nat_iu = jnp.sqrt(jnp.sum(attiu[...] ** 2))
        m_ui[...] = jnp.full((1, 128), nat_ui * (jnp.sqrt(mx[0]) + jnp.sqrt(mx[1])))
        m_iu[...] = jnp.full((1, 128), nat_iu * (jnp.sqrt(mx[2]) + jnp.sqrt(mx[3])))


def _tc_linear(x_user, x_item, Wl_ui, bl_ui, Wr_ui, br_ui, att_ui,
               Wl_iu, bl_iu, Wr_iu, br_iu, att_iu):
    grid = N // _TC_BLOCK
    row_spec = pl.BlockSpec((_TC_BLOCK, D), lambda i: (i, 0))
    w_spec = pl.BlockSpec((D, C), lambda i: (0, 0))
    v_spec = pl.BlockSpec((C,), lambda i: (0,))
    m_spec = pl.BlockSpec((1, 128), lambda i: (0, 0))
    return pl.pallas_call(
        _tc_body,
        grid=(grid,),
        in_specs=[row_spec, row_spec, w_spec, v_spec, w_spec, v_spec, v_spec,
                  w_spec, v_spec, w_spec, v_spec, v_spec],
        out_specs=[row_spec, row_spec, row_spec, row_spec, m_spec, m_spec],
        out_shape=[
            jax.ShapeDtypeStruct((N, C), jnp.float32),
            jax.ShapeDtypeStruct((N, C), jnp.float32),
            jax.ShapeDtypeStruct((N, C), jnp.float32),
            jax.ShapeDtypeStruct((N, C), jnp.float32),
            jax.ShapeDtypeStruct((1, 128), jnp.float32),
            jax.ShapeDtypeStruct((1, 128), jnp.float32),
        ],
        scratch_shapes=[pltpu.SMEM((4,), jnp.float32)],
    )(x_user, x_item, Wl_ui, bl_ui, Wr_ui, br_ui, att_ui,
      Wl_iu, bl_iu, Wr_iu, br_iu, att_iu)


def _edge_pass(hs_h, hd_h, src_h, dst_h, att_h, bias_h, m_h, out_h,
               c, s, acc, den_sh, src_v, dst_v, hs_rows, hd_rows, tbuf,
               pbuf, att_v, bias_v, m_buf, row_buf, denbuf, sem_a, sem_b):
    pltpu.sync_copy(att_h, att_v)
    pltpu.sync_copy(bias_h, bias_v)
    pltpu.sync_copy(m_h.at[0], m_buf)
    att_regs = [att_v[pl.ds(k * L, L)] for k in range(C // L)]
    bias_regs = [bias_v[pl.ds(k * L, L)] for k in range(C // L)]
    m16 = m_buf[pl.ds(0, L)]

    @pl.loop(0, RCHUNK)
    def _(i):
        for k in range(C // L):
            row_buf[i, pl.ds(k * L, L)] = jnp.zeros((L,), jnp.float32)

    @pl.loop(0, DCHUNK // L)
    def _(i):
        denbuf[pl.ds(i * L, L)] = jnp.zeros((L,), jnp.float32)

    @pl.loop(0, ROWS_PER_TEC // RCHUNK)
    def _(j):
        pltpu.sync_copy(row_buf, acc.at[pl.ds(s * ROWS_PER_TEC + j * RCHUNK, RCHUNK)])

    @pl.loop(0, ROWS_PER_TEC // DCHUNK)
    def _(j):
        pltpu.sync_copy(denbuf, den_sh.at[pl.ds(s * ROWS_PER_TEC + j * DCHUNK, DCHUNK)])

    plsc.subcore_barrier()

    nrem = NBLK - (NBLK // NS) * NS
    nblk = jnp.where(s < nrem, NBLK // NS + 1, NBLK // NS)

    @pl.loop(0, nblk)
    def _(i):
        base = pl.multiple_of((i * NS + s) * B, B)
        pltpu.sync_copy(src_h.at[pl.ds(base, B)], src_v)
        pltpu.sync_copy(dst_h.at[pl.ds(base, B)], dst_v)
        g1 = pltpu.async_copy(hs_h.at[src_v], hs_rows, sem_a)
        g2 = pltpu.async_copy(hd_h.at[dst_v], hd_rows, sem_b)
        g1.wait()
        g2.wait()

        @pl.loop(0, B // L)
        def _(g):
            e0 = g * L
            for e in range(L):
                t = jnp.zeros((L,), jnp.float32)
                for k in range(C // L):
                    z = (hs_rows[e0 + e, pl.ds(k * L, L)]
                         + hd_rows[e0 + e, pl.ds(k * L, L)])
                    z = jnp.maximum(z, 0.2 * z)
                    t = t + att_regs[k] * z
                tbuf[pl.ds(e * L, L)] = t
            rows16 = lax.iota(jnp.int32, L) * L
            lsum = jnp.zeros((L,), jnp.float32)
            for j in range(L):
                lsum = lsum + plsc.load_gather(tbuf, [rows16 + j])
            p16 = jnp.exp(lsum - m16)
            pbuf[pl.ds(e0, L)] = p16

            for e in range(L):
                pe = p16[e]
                for k in range(C // L):
                    hs_rows[e0 + e, pl.ds(k * L, L)] = (
                        hs_rows[e0 + e, pl.ds(k * L, L)] * pe)

        pltpu.sync_copy(pbuf, den_sh.at[dst_v], add=True)
        pltpu.sync_copy(hs_rows, acc.at[dst_v], add=True)

    plsc.subcore_barrier()

    @pl.loop(0, ROWS_PER_TEC // DCHUNK)
    def _(j):
        row0 = s * ROWS_PER_TEC + j * DCHUNK
        pltpu.sync_copy(den_sh.at[pl.ds(row0, DCHUNK)], denbuf)

        @pl.loop(0, DCHUNK // RCHUNK)
        def _(r2):
            row0b = row0 + r2 * RCHUNK

            @pl.when(row0b < N)
            def _():
                pltpu.sync_copy(acc.at[pl.ds(row0b, RCHUNK)], row_buf)
                invv = 1.0 / (denbuf[pl.ds(r2 * RCHUNK, L)] + 1e-16)
                for r in range(RCHUNK):
                    ir = invv[r]
                    for k in range(C // L):
                        row_buf[r, pl.ds(k * L, L)] = (
                            row_buf[r, pl.ds(k * L, L)] * ir + bias_regs[k])
                pltpu.sync_copy(row_buf, out_h.at[pl.ds(row0b, RCHUNK)])


@functools.partial(
    pl.kernel,
    out_type=(jax.ShapeDtypeStruct((N, C), jnp.float32),
              jax.ShapeDtypeStruct((N, C), jnp.float32)),
    mesh=plsc.VectorSubcoreMesh(core_axis_name="c", subcore_axis_name="s",
                                num_cores=NC, num_subcores=NS),
    compiler_params=pltpu.CompilerParams(needs_layout_passes=False),
    scratch_types=[
        pltpu.VMEM_SHARED((N_PAD, C), jnp.float32),
        pltpu.VMEM_SHARED((N_PAD,), jnp.float32),
        pltpu.VMEM((B,), jnp.int32),
        pltpu.VMEM((B,), jnp.int32),
        pltpu.VMEM((B, C), jnp.float32),
        pltpu.VMEM((B, C), jnp.float32),
        pltpu.VMEM((L * L,), jnp.float32),
        pltpu.VMEM((B,), jnp.float32),
        pltpu.VMEM((C,), jnp.float32),
        pltpu.VMEM((C,), jnp.float32),
        pltpu.VMEM((128,), jnp.float32),
        pltpu.VMEM((RCHUNK, C), jnp.float32),
        pltpu.VMEM((DCHUNK,), jnp.float32),
        pltpu.SemaphoreType.DMA,
        pltpu.SemaphoreType.DMA,
    ],
)
def _sc_edges(hs_ui, hd_ui, src_ui, dst_ui, att_ui, bias_ui, m_ui,
              hs_iu, hd_iu, src_iu, dst_iu, att_iu, bias_iu, m_iu,
              out_user, out_item,
              acc, den_sh, src_v, dst_v, hs_rows, hd_rows, tbuf, pbuf,
              att_v, bias_v, m_buf, row_buf, denbuf, sem_a, sem_b):
    c = lax.axis_index("c")
    s = lax.axis_index("s")
    scratch = (acc, den_sh, src_v, dst_v, hs_rows, hd_rows, tbuf, pbuf,
               att_v, bias_v, m_buf, row_buf, denbuf, sem_a, sem_b)

    @pl.when(c == 0)
    def _():
        _edge_pass(hs_ui, hd_ui, src_ui, dst_ui, att_ui, bias_ui, m_ui,
                   out_item, c, s, *scratch)

    @pl.when(c == 1)
    def _():
        _edge_pass(hs_iu, hd_iu, src_iu, dst_iu, att_iu, bias_iu, m_iu,
                   out_user, c, s, *scratch)


def kernel(x_user, x_item, ei_user_item, ei_item_user,
           Wl_ui, bl_ui, Wr_ui, br_ui, att_ui, bias_ui,
           Wl_iu, bl_iu, Wr_iu, br_iu, att_iu, bias_iu):
    hs_ui, hd_ui, hs_iu, hd_iu, m_ui, m_iu = _tc_linear(
        x_user, x_item, Wl_ui, bl_ui, Wr_ui, br_ui, att_ui,
        Wl_iu, bl_iu, Wr_iu, br_iu, att_iu)
    src_ui = ei_user_item[0].astype(jnp.int32)
    dst_ui = ei_user_item[1].astype(jnp.int32)
    src_iu = ei_item_user[0].astype(jnp.int32)
    dst_iu = ei_item_user[1].astype(jnp.int32)
    out_user, out_item = _sc_edges(
        hs_ui, hd_ui, src_ui, dst_ui, att_ui, bias_ui, m_ui,
        hs_iu, hd_iu, src_iu, dst_iu, att_iu, bias_iu, m_iu)
    return (out_user, out_item)


# speedup vs baseline: 15.5816x; 1.5249x over previous
import functools

import jax
import jax.numpy as jnp
from jax import lax
from jax.experimental import pallas as pl
from jax.experimental.pallas import tpu as pltpu
from jax.experimental.pallas import tpu_sc as plsc

N = 10000
D = 128
C = 128
E = 320000

NC = 2
NS = 16
L = 16

B = 80
NBLK = E // B
NPT = NBLK // NS
N_PAD = 10240
ROWS_PER_TEC = N_PAD // NS
RCHUNK = 16
DCHUNK = 128

_TC_BLOCK = 1000


def _tc_body(xu, xi, wlui, blui, wrui, brui, attui, wliu, bliu, wriu, briu,
             attiu, hs_ui, hd_ui, hs_iu, hd_iu, m_ui, m_iu, mx):
    step = pl.program_id(0)

    @pl.when(step == 0)
    def _():
        for i in range(4):
            mx[i] = 0.0

    a = jnp.dot(xu[...], wlui[...], preferred_element_type=jnp.float32) + blui[...]
    b = jnp.dot(xi[...], wrui[...], preferred_element_type=jnp.float32) + brui[...]
    c = jnp.dot(xi[...], wliu[...], preferred_element_type=jnp.float32) + bliu[...]
    d = jnp.dot(xu[...], wriu[...], preferred_element_type=jnp.float32) + briu[...]
    hs_ui[...] = a
    hd_ui[...] = b
    hs_iu[...] = c
    hd_iu[...] = d
    mx[0] = jnp.maximum(mx[0], jnp.max(jnp.sum(a * a, axis=1)))
    mx[1] = jnp.maximum(mx[1], jnp.max(jnp.sum(b * b, axis=1)))
    mx[2] = jnp.maximum(mx[2], jnp.max(jnp.sum(c * c, axis=1)))
    mx[3] = jnp.maximum(mx[3], jnp.max(jnp.sum(d * d, axis=1)))

    @pl.when(step == pl.num_programs(0) - 1)
    def _():
        nat_ui = jnp.sqrt(jnp.sum(attui[...] ** 2))
        nat_iu = jnp.sqrt(jnp.sum(attiu[...] ** 2))
        m_ui[...] = jnp.full((1, 128), nat_ui * (jnp.sqrt(mx[0]) + jnp.sqrt(mx[1])))
        m_iu[...] = jnp.full((1, 128), nat_iu * (jnp.sqrt(mx[2]) + jnp.sqrt(mx[3])))


def _tc_linear(x_user, x_item, Wl_ui, bl_ui, Wr_ui, br_ui, att_ui,
               Wl_iu, bl_iu, Wr_iu, br_iu, att_iu):
    grid = N // _TC_BLOCK
    row_spec = pl.BlockSpec((_TC_BLOCK, D), lambda i: (i, 0))
    w_spec = pl.BlockSpec((D, C), lambda i: (0, 0))
    v_spec = pl.BlockSpec((C,), lambda i: (0,))
    m_spec = pl.BlockSpec((1, 128), lambda i: (0, 0))
    return pl.pallas_call(
        _tc_body,
        grid=(grid,),
        in_specs=[row_spec, row_spec, w_spec, v_spec, w_spec, v_spec, v_spec,
                  w_spec, v_spec, w_spec, v_spec, v_spec],
        out_specs=[row_spec, row_spec, row_spec, row_spec, m_spec, m_spec],
        out_shape=[
            jax.ShapeDtypeStruct((N, C), jnp.float32),
            jax.ShapeDtypeStruct((N, C), jnp.float32),
            jax.ShapeDtypeStruct((N, C), jnp.float32),
            jax.ShapeDtypeStruct((N, C), jnp.float32),
            jax.ShapeDtypeStruct((1, 128), jnp.float32),
            jax.ShapeDtypeStruct((1, 128), jnp.float32),
        ],
        scratch_shapes=[pltpu.SMEM((4,), jnp.float32)],
    )(x_user, x_item, Wl_ui, bl_ui, Wr_ui, br_ui, att_ui,
      Wl_iu, bl_iu, Wr_iu, br_iu, att_iu)


def _edge_pass(hs_h, hd_h, src_h, dst_h, att_h, bias_h, m_h, out_h,
               c, s, acc, den_sh,
               src_v0, dst_v0, src_v1, dst_v1,
               hs_rows0, hd_rows0, hs_rows1, hd_rows1, tbuf,
               pbuf, att_v, bias_v, m_buf, row_buf, denbuf,
               sem_i0, sem_i1, sem_a0, sem_b0, sem_a1, sem_b1):
    pltpu.sync_copy(att_h, att_v)
    pltpu.sync_copy(bias_h, bias_v)
    pltpu.sync_copy(m_h.at[0], m_buf)
    att_regs = [att_v[pl.ds(k * L, L)] for k in range(C // L)]
    bias_regs = [bias_v[pl.ds(k * L, L)] for k in range(C // L)]
    m16 = m_buf[pl.ds(0, L)]

    @pl.loop(0, RCHUNK)
    def _(i):
        for k in range(C // L):
            row_buf[i, pl.ds(k * L, L)] = jnp.zeros((L,), jnp.float32)

    @pl.loop(0, DCHUNK // L)
    def _(i):
        denbuf[pl.ds(i * L, L)] = jnp.zeros((L,), jnp.float32)

    @pl.loop(0, ROWS_PER_TEC // RCHUNK)
    def _(j):
        pltpu.sync_copy(row_buf, acc.at[pl.ds(s * ROWS_PER_TEC + j * RCHUNK, RCHUNK)])

    @pl.loop(0, ROWS_PER_TEC // DCHUNK)
    def _(j):
        pltpu.sync_copy(denbuf, den_sh.at[pl.ds(s * ROWS_PER_TEC + j * DCHUNK, DCHUNK)])

    plsc.subcore_barrier()

    ibufs = [(src_v0, dst_v0, sem_i0), (src_v1, dst_v1, sem_i1)]
    rbufs = [(hs_rows0, hd_rows0, sem_a0, sem_b0),
             (hs_rows1, hd_rows1, sem_a1, sem_b1)]

    def idx_start(blk, p):
        base = pl.multiple_of((blk * NS + s) * B, B)
        sv, dv, si = ibufs[p]
        pltpu.async_copy(src_h.at[pl.ds(base, B)], sv, si)
        pltpu.async_copy(dst_h.at[pl.ds(base, B)], dv, si)

    def idx_wait(p):
        sv, dv, si = ibufs[p]
        pltpu.make_async_copy(src_h.at[pl.ds(0, B)], sv, si).wait()
        pltpu.make_async_copy(dst_h.at[pl.ds(0, B)], dv, si).wait()

    def rows_start(p):
        sv, dv, _ = ibufs[p]
        hsb, hdb, sa, sb = rbufs[p]
        pltpu.async_copy(hs_h.at[sv], hsb, sa)
        pltpu.async_copy(hd_h.at[dv], hdb, sb)

    def rows_wait(p):
        sv, dv, _ = ibufs[p]
        hsb, hdb, sa, sb = rbufs[p]
        pltpu.make_async_copy(hs_h.at[sv], hsb, sa).wait()
        pltpu.make_async_copy(hd_h.at[dv], hdb, sb).wait()

    def compute_scatter(p):
        _, dv, _ = ibufs[p]
        hsb, hdb, _, _ = rbufs[p]

        @pl.loop(0, B // L)
        def _(g):
            e0 = g * L
            for e in range(L):
                t = jnp.zeros((L,), jnp.float32)
                for k in range(C // L):
                    z = (hsb[e0 + e, pl.ds(k * L, L)]
                         + hdb[e0 + e, pl.ds(k * L, L)])
                    z = jnp.maximum(z, 0.2 * z)
                    t = t + att_regs[k] * z
                tbuf[pl.ds(e * L, L)] = t
            rows16 = lax.iota(jnp.int32, L) * L
            lsum = jnp.zeros((L,), jnp.float32)
            for j in range(L):
                lsum = lsum + plsc.load_gather(tbuf, [rows16 + j])
            p16 = jnp.exp(lsum - m16)
            pbuf[pl.ds(e0, L)] = p16

            for e in range(L):
                pe = p16[e]
                for k in range(C // L):
                    hsb[e0 + e, pl.ds(k * L, L)] = hsb[e0 + e, pl.ds(k * L, L)] * pe

        pltpu.sync_copy(pbuf, den_sh.at[dv], add=True)
        pltpu.sync_copy(hsb, acc.at[dv], add=True)

    idx_start(0, 0)
    idx_wait(0)
    rows_start(0)
    idx_start(1, 1)

    @pl.loop(0, NPT, step=2)
    def _(blk0):
        for b in range(2):
            blk = blk0 + b
            rows_wait(b)

            @pl.when(blk + 1 < NPT)
            def _():
                idx_wait(1 - b)
                rows_start(1 - b)

            compute_scatter(b)

            @pl.when(blk + 2 < NPT)
            def _():
                idx_start(blk + 2, b)

    plsc.subcore_barrier()

    @pl.loop(0, ROWS_PER_TEC // DCHUNK)
    def _(j):
        row0 = s * ROWS_PER_TEC + j * DCHUNK
        pltpu.sync_copy(den_sh.at[pl.ds(row0, DCHUNK)], denbuf)

        @pl.loop(0, DCHUNK // RCHUNK)
        def _(r2):
            row0b = row0 + r2 * RCHUNK

            @pl.when(row0b < N)
            def _():
                pltpu.sync_copy(acc.at[pl.ds(row0b, RCHUNK)], row_buf)
                invv = 1.0 / (denbuf[pl.ds(r2 * RCHUNK, L)] + 1e-16)
                for r in range(RCHUNK):
                    ir = invv[r]
                    for k in range(C // L):
                        row_buf[r, pl.ds(k * L, L)] = (
                            row_buf[r, pl.ds(k * L, L)] * ir + bias_regs[k])
                pltpu.sync_copy(row_buf, out_h.at[pl.ds(row0b, RCHUNK)])


@functools.partial(
    pl.kernel,
    out_type=(jax.ShapeDtypeStruct((N, C), jnp.float32),
              jax.ShapeDtypeStruct((N, C), jnp.float32)),
    mesh=plsc.VectorSubcoreMesh(core_axis_name="c", subcore_axis_name="s",
                                num_cores=NC, num_subcores=NS),
    compiler_params=pltpu.CompilerParams(needs_layout_passes=False),
    scratch_types=[
        pltpu.VMEM_SHARED((N_PAD, C), jnp.float32),
        pltpu.VMEM_SHARED((N_PAD,), jnp.float32),
        pltpu.VMEM((B,), jnp.int32),
        pltpu.VMEM((B,), jnp.int32),
        pltpu.VMEM((B,), jnp.int32),
        pltpu.VMEM((B,), jnp.int32),
        pltpu.VMEM((B, C), jnp.float32),
        pltpu.VMEM((B, C), jnp.float32),
        pltpu.VMEM((B, C), jnp.float32),
        pltpu.VMEM((B, C), jnp.float32),
        pltpu.VMEM((L * L,), jnp.float32),
        pltpu.VMEM((B,), jnp.float32),
        pltpu.VMEM((C,), jnp.float32),
        pltpu.VMEM((C,), jnp.float32),
        pltpu.VMEM((128,), jnp.float32),
        pltpu.VMEM((RCHUNK, C), jnp.float32),
        pltpu.VMEM((DCHUNK,), jnp.float32),
        pltpu.SemaphoreType.DMA,
        pltpu.SemaphoreType.DMA,
        pltpu.SemaphoreType.DMA,
        pltpu.SemaphoreType.DMA,
        pltpu.SemaphoreType.DMA,
        pltpu.SemaphoreType.DMA,
    ],
)
def _sc_edges(hs_ui, hd_ui, src_ui, dst_ui, att_ui, bias_ui, m_ui,
              hs_iu, hd_iu, src_iu, dst_iu, att_iu, bias_iu, m_iu,
              out_user, out_item,
              acc, den_sh, src_v0, dst_v0, src_v1, dst_v1,
              hs_rows0, hd_rows0, hs_rows1, hd_rows1, tbuf, pbuf,
              att_v, bias_v, m_buf, row_buf, denbuf,
              sem_i0, sem_i1, sem_a0, sem_b0, sem_a1, sem_b1):
    c = lax.axis_index("c")
    s = lax.axis_index("s")
    scratch = (acc, den_sh, src_v0, dst_v0, src_v1, dst_v1,
               hs_rows0, hd_rows0, hs_rows1, hd_rows1, tbuf, pbuf,
               att_v, bias_v, m_buf, row_buf, denbuf,
               sem_i0, sem_i1, sem_a0, sem_b0, sem_a1, sem_b1)

    @pl.when(c == 0)
    def _():
        _edge_pass(hs_ui, hd_ui, src_ui, dst_ui, att_ui, bias_ui, m_ui,
                   out_item, c, s, *scratch)

    @pl.when(c == 1)
    def _():
        _edge_pass(hs_iu, hd_iu, src_iu, dst_iu, att_iu, bias_iu, m_iu,
                   out_user, c, s, *scratch)


def kernel(x_user, x_item, ei_user_item, ei_item_user,
           Wl_ui, bl_ui, Wr_ui, br_ui, att_ui, bias_ui,
           Wl_iu, bl_iu, Wr_iu, br_iu, att_iu, bias_iu):
    hs_ui, hd_ui, hs_iu, hd_iu, m_ui, m_iu = _tc_linear(
        x_user, x_item, Wl_ui, bl_ui, Wr_ui, br_ui, att_ui,
        Wl_iu, bl_iu, Wr_iu, br_iu, att_iu)
    src_ui = ei_user_item[0].astype(jnp.int32)
    dst_ui = ei_user_item[1].astype(jnp.int32)
    src_iu = ei_item_user[0].astype(jnp.int32)
    dst_iu = ei_item_user[1].astype(jnp.int32)
    out_user, out_item = _sc_edges(
        hs_ui, hd_ui, src_ui, dst_ui, att_ui, bias_ui, m_ui,
        hs_iu, hd_iu, src_iu, dst_iu, att_iu, bias_iu, m_iu)
    return (out_user, out_item)


# speedup vs baseline: 18.1634x; 1.1657x over previous
import functools

import jax
import jax.numpy as jnp
from jax import lax
from jax.experimental import pallas as pl
from jax.experimental.pallas import tpu as pltpu
from jax.experimental.pallas import tpu_sc as plsc

N = 10000
D = 128
C = 128
E = 320000

NC = 2
NS = 16
L = 16

B = 80
NBLK = E // B
NPT = NBLK // NS
N_PAD = 10240
ROWS_PER_TEC = N_PAD // NS
RCHUNK = 16
DCHUNK = 128

_TC_BLOCK = 1000


def _tc_body(xu, xi, wlui, blui, wrui, brui, attui, wliu, bliu, wriu, briu,
             attiu, hs_ui, hd_ui, hs_iu, hd_iu, m_ui, m_iu, mx):
    step = pl.program_id(0)

    @pl.when(step == 0)
    def _():
        for i in range(4):
            mx[i] = 0.0

    a = jnp.dot(xu[...], wlui[...], preferred_element_type=jnp.float32) + blui[...]
    b = jnp.dot(xi[...], wrui[...], preferred_element_type=jnp.float32) + brui[...]
    c = jnp.dot(xi[...], wliu[...], preferred_element_type=jnp.float32) + bliu[...]
    d = jnp.dot(xu[...], wriu[...], preferred_element_type=jnp.float32) + briu[...]
    hs_ui[...] = a
    hd_ui[...] = b
    hs_iu[...] = c
    hd_iu[...] = d
    mx[0] = jnp.maximum(mx[0], jnp.max(jnp.sum(a * a, axis=1)))
    mx[1] = jnp.maximum(mx[1], jnp.max(jnp.sum(b * b, axis=1)))
    mx[2] = jnp.maximum(mx[2], jnp.max(jnp.sum(c * c, axis=1)))
    mx[3] = jnp.maximum(mx[3], jnp.max(jnp.sum(d * d, axis=1)))

    @pl.when(step == pl.num_programs(0) - 1)
    def _():
        nat_ui = jnp.sqrt(jnp.sum(attui[...] ** 2))
        nat_iu = jnp.sqrt(jnp.sum(attiu[...] ** 2))
        m_ui[...] = jnp.full((1, 128), nat_ui * (jnp.sqrt(mx[0]) + jnp.sqrt(mx[1])))
        m_iu[...] = jnp.full((1, 128), nat_iu * (jnp.sqrt(mx[2]) + jnp.sqrt(mx[3])))


def _tc_linear(x_user, x_item, Wl_ui, bl_ui, Wr_ui, br_ui, att_ui,
               Wl_iu, bl_iu, Wr_iu, br_iu, att_iu):
    grid = N // _TC_BLOCK
    row_spec = pl.BlockSpec((_TC_BLOCK, D), lambda i: (i, 0))
    w_spec = pl.BlockSpec((D, C), lambda i: (0, 0))
    v_spec = pl.BlockSpec((C,), lambda i: (0,))
    m_spec = pl.BlockSpec((1, 128), lambda i: (0, 0))
    return pl.pallas_call(
        _tc_body,
        grid=(grid,),
        in_specs=[row_spec, row_spec, w_spec, v_spec, w_spec, v_spec, v_spec,
                  w_spec, v_spec, w_spec, v_spec, v_spec],
        out_specs=[row_spec, row_spec, row_spec, row_spec, m_spec, m_spec],
        out_shape=[
            jax.ShapeDtypeStruct((N, C), jnp.float32),
            jax.ShapeDtypeStruct((N, C), jnp.float32),
            jax.ShapeDtypeStruct((N, C), jnp.float32),
            jax.ShapeDtypeStruct((N, C), jnp.float32),
            jax.ShapeDtypeStruct((1, 128), jnp.float32),
            jax.ShapeDtypeStruct((1, 128), jnp.float32),
        ],
        scratch_shapes=[pltpu.SMEM((4,), jnp.float32)],
    )(x_user, x_item, Wl_ui, bl_ui, Wr_ui, br_ui, att_ui,
      Wl_iu, bl_iu, Wr_iu, br_iu, att_iu)


def _edge_pass(hs_h, hd_h, src_h, dst_h, att_h, bias_h, m_h, out_h,
               c, s, acc, den_sh,
               src_v0, dst_v0, src_v1, dst_v1,
               hs_rows0, hd_rows0, hs_rows1, hd_rows1, tbuf,
               pbuf0, pbuf1, dvv0, dvv1, att_v, bias_v, m_buf, row_buf, denbuf,
               sem_i0, sem_i1, sem_a0, sem_b0, sem_a1, sem_b1,
               sem_s0, sem_s1):
    pltpu.sync_copy(att_h, att_v)
    pltpu.sync_copy(bias_h, bias_v)
    pltpu.sync_copy(m_h.at[0], m_buf)
    att_regs = [att_v[pl.ds(k * L, L)] for k in range(C // L)]
    bias_regs = [bias_v[pl.ds(k * L, L)] for k in range(C // L)]
    m16 = m_buf[pl.ds(0, L)]

    @pl.loop(0, RCHUNK)
    def _(i):
        for k in range(C // L):
            row_buf[i, pl.ds(k * L, L)] = jnp.zeros((L,), jnp.float32)

    @pl.loop(0, DCHUNK // L)
    def _(i):
        denbuf[pl.ds(i * L, L)] = jnp.zeros((L,), jnp.float32)

    @pl.loop(0, ROWS_PER_TEC // RCHUNK)
    def _(j):
        pltpu.sync_copy(row_buf, acc.at[pl.ds(s * ROWS_PER_TEC + j * RCHUNK, RCHUNK)])

    @pl.loop(0, ROWS_PER_TEC // DCHUNK)
    def _(j):
        pltpu.sync_copy(denbuf, den_sh.at[pl.ds(s * ROWS_PER_TEC + j * DCHUNK, DCHUNK)])

    plsc.subcore_barrier()

    ibufs = [(src_v0, dst_v0, sem_i0), (src_v1, dst_v1, sem_i1)]
    rbufs = [(hs_rows0, hd_rows0, sem_a0, sem_b0),
             (hs_rows1, hd_rows1, sem_a1, sem_b1)]
    sbufs = [(dvv0, pbuf0, sem_s0), (dvv1, pbuf1, sem_s1)]

    def idx_start(blk, p):
        base = pl.multiple_of((blk * NS + s) * B, B)
        sv, dv, si = ibufs[p]
        pltpu.async_copy(src_h.at[pl.ds(base, B)], sv, si)
        pltpu.async_copy(dst_h.at[pl.ds(base, B)], dv, si)

    def idx_wait(p):
        sv, dv, si = ibufs[p]
        pltpu.make_async_copy(src_h.at[pl.ds(0, B)], sv, si).wait()
        pltpu.make_async_copy(dst_h.at[pl.ds(0, B)], dv, si).wait()

    def rows_start(p):
        sv, dv, _ = ibufs[p]
        hsb, hdb, sa, sb = rbufs[p]
        pltpu.async_copy(hs_h.at[sv], hsb, sa)
        pltpu.async_copy(hd_h.at[dv], hdb, sb)

    def rows_wait(p):
        sv, dv, _ = ibufs[p]
        hsb, hdb, sa, sb = rbufs[p]
        pltpu.make_async_copy(hs_h.at[sv], hsb, sa).wait()
        pltpu.make_async_copy(hd_h.at[dv], hdb, sb).wait()

    def compute_scatter(p):
        _, dv, _ = ibufs[p]
        hsb, hdb, _, _ = rbufs[p]
        dvv, pbuf, sem_s = sbufs[p]

        @pl.loop(0, B // L)
        def _(g):
            e0 = g * L
            for e in range(L):
                t = jnp.zeros((L,), jnp.float32)
                for k in range(C // L):
                    z = (hsb[e0 + e, pl.ds(k * L, L)]
                         + hdb[e0 + e, pl.ds(k * L, L)])
                    z = jnp.maximum(z, 0.2 * z)
                    t = t + att_regs[k] * z
                tbuf[pl.ds(e * L, L)] = t
            rows16 = lax.iota(jnp.int32, L) * L
            lsum = jnp.zeros((L,), jnp.float32)
            for j in range(L):
                lsum = lsum + plsc.load_gather(tbuf, [rows16 + j])
            p16 = jnp.exp(lsum - m16)
            pbuf[pl.ds(e0, L)] = p16

            for e in range(L):
                pe = p16[e]
                for k in range(C // L):
                    hsb[e0 + e, pl.ds(k * L, L)] = hsb[e0 + e, pl.ds(k * L, L)] * pe

        for k in range(B // L):
            dvv[pl.ds(k * L, L)] = dv[pl.ds(k * L, L)]
        pltpu.async_copy(pbuf, den_sh.at[dvv], sem_s, add=True)
        pltpu.async_copy(hsb, acc.at[dvv], sem_s, add=True)

    def scatter_wait(p):
        dvv, pbuf, sem_s = sbufs[p]
        hsb, _, _, _ = rbufs[p]
        pltpu.make_async_copy(pbuf, den_sh.at[dvv], sem_s).wait()
        pltpu.make_async_copy(hsb, acc.at[dvv], sem_s).wait()

    idx_start(0, 0)
    idx_wait(0)
    rows_start(0)
    idx_start(1, 1)

    @pl.loop(0, NPT, step=2)
    def _(blk0):
        for b in range(2):
            blk = blk0 + b
            rows_wait(b)

            @pl.when(blk + 1 < NPT)
            def _():
                idx_wait(1 - b)

                @pl.when(blk >= 1)
                def _():
                    scatter_wait(1 - b)

                rows_start(1 - b)

            compute_scatter(b)

            @pl.when(blk + 2 < NPT)
            def _():
                idx_start(blk + 2, b)

    scatter_wait(0)
    scatter_wait(1)
    plsc.subcore_barrier()

    @pl.loop(0, ROWS_PER_TEC // DCHUNK)
    def _(j):
        row0 = s * ROWS_PER_TEC + j * DCHUNK
        pltpu.sync_copy(den_sh.at[pl.ds(row0, DCHUNK)], denbuf)

        @pl.loop(0, DCHUNK // RCHUNK)
        def _(r2):
            row0b = row0 + r2 * RCHUNK

            @pl.when(row0b < N)
            def _():
                pltpu.sync_copy(acc.at[pl.ds(row0b, RCHUNK)], row_buf)
                invv = 1.0 / (denbuf[pl.ds(r2 * RCHUNK, L)] + 1e-16)
                for r in range(RCHUNK):
                    ir = invv[r]
                    for k in range(C // L):
                        row_buf[r, pl.ds(k * L, L)] = (
                            row_buf[r, pl.ds(k * L, L)] * ir + bias_regs[k])
                pltpu.sync_copy(row_buf, out_h.at[pl.ds(row0b, RCHUNK)])


@functools.partial(
    pl.kernel,
    out_type=(jax.ShapeDtypeStruct((N, C), jnp.float32),
              jax.ShapeDtypeStruct((N, C), jnp.float32)),
    mesh=plsc.VectorSubcoreMesh(core_axis_name="c", subcore_axis_name="s",
                                num_cores=NC, num_subcores=NS),
    compiler_params=pltpu.CompilerParams(needs_layout_passes=False),
    scratch_types=[
        pltpu.VMEM_SHARED((N_PAD, C), jnp.float32),
        pltpu.VMEM_SHARED((N_PAD,), jnp.float32),
        pltpu.VMEM((B,), jnp.int32),
        pltpu.VMEM((B,), jnp.int32),
        pltpu.VMEM((B,), jnp.int32),
        pltpu.VMEM((B,), jnp.int32),
        pltpu.VMEM((B, C), jnp.float32),
        pltpu.VMEM((B, C), jnp.float32),
        pltpu.VMEM((B, C), jnp.float32),
        pltpu.VMEM((B, C), jnp.float32),
        pltpu.VMEM((L * L,), jnp.float32),
        pltpu.VMEM((B,), jnp.float32),
        pltpu.VMEM((B,), jnp.float32),
        pltpu.VMEM((B,), jnp.int32),
        pltpu.VMEM((B,), jnp.int32),
        pltpu.VMEM((C,), jnp.float32),
        pltpu.VMEM((C,), jnp.float32),
        pltpu.VMEM((128,), jnp.float32),
        pltpu.VMEM((RCHUNK, C), jnp.float32),
        pltpu.VMEM((DCHUNK,), jnp.float32),
        pltpu.SemaphoreType.DMA,
        pltpu.SemaphoreType.DMA,
        pltpu.SemaphoreType.DMA,
        pltpu.SemaphoreType.DMA,
        pltpu.SemaphoreType.DMA,
        pltpu.SemaphoreType.DMA,
        pltpu.SemaphoreType.DMA,
        pltpu.SemaphoreType.DMA,
    ],
)
def _sc_edges(hs_ui, hd_ui, src_ui, dst_ui, att_ui, bias_ui, m_ui,
              hs_iu, hd_iu, src_iu, dst_iu, att_iu, bias_iu, m_iu,
              out_user, out_item,
              acc, den_sh, src_v0, dst_v0, src_v1, dst_v1,
              hs_rows0, hd_rows0, hs_rows1, hd_rows1, tbuf,
              pbuf0, pbuf1, dvv0, dvv1,
              att_v, bias_v, m_buf, row_buf, denbuf,
              sem_i0, sem_i1, sem_a0, sem_b0, sem_a1, sem_b1,
              sem_s0, sem_s1):
    c = lax.axis_index("c")
    s = lax.axis_index("s")
    scratch = (acc, den_sh, src_v0, dst_v0, src_v1, dst_v1,
               hs_rows0, hd_rows0, hs_rows1, hd_rows1, tbuf,
               pbuf0, pbuf1, dvv0, dvv1,
               att_v, bias_v, m_buf, row_buf, denbuf,
               sem_i0, sem_i1, sem_a0, sem_b0, sem_a1, sem_b1,
               sem_s0, sem_s1)

    @pl.when(c == 0)
    def _():
        _edge_pass(hs_ui, hd_ui, src_ui, dst_ui, att_ui, bias_ui, m_ui,
                   out_item, c, s, *scratch)

    @pl.when(c == 1)
    def _():
        _edge_pass(hs_iu, hd_iu, src_iu, dst_iu, att_iu, bias_iu, m_iu,
                   out_user, c, s, *scratch)


def kernel(x_user, x_item, ei_user_item, ei_item_user,
           Wl_ui, bl_ui, Wr_ui, br_ui, att_ui, bias_ui,
           Wl_iu, bl_iu, Wr_iu, br_iu, att_iu, bias_iu):
    hs_ui, hd_ui, hs_iu, hd_iu, m_ui, m_iu = _tc_linear(
        x_user, x_item, Wl_ui, bl_ui, Wr_ui, br_ui, att_ui,
        Wl_iu, bl_iu, Wr_iu, br_iu, att_iu)
    src_ui = ei_user_item[0].astype(jnp.int32)
    dst_ui = ei_user_item[1].astype(jnp.int32)
    src_iu = ei_item_user[0].astype(jnp.int32)
    dst_iu = ei_item_user[1].astype(jnp.int32)
    out_user, out_item = _sc_edges(
        hs_ui, hd_ui, src_ui, dst_ui, att_ui, bias_ui, m_ui,
        hs_iu, hd_iu, src_iu, dst_iu, att_iu, bias_iu, m_iu)
    return (out_user, out_item)
